# Initial kernel scaffold; baseline (speedup 1.0000x reference)
import jax
import jax.numpy as jnp
from jax.experimental import pallas as pl


def kernel(h, edge_index, e, snorm_n, W, b, gamma, beta):
    raise NotImplementedError("write your pallas kernel here")


# speedup vs baseline: 6.7916x; 6.7916x over previous
import functools

import jax
import jax.numpy as jnp
from jax import lax
from jax.experimental import pallas as pl
from jax.experimental.pallas import tpu as pltpu
from jax.experimental.pallas import tpu_sc as plsc

N_NODES = 100000
N_EDGES = 3200000
DIM = 25
DP = 32
BINSZ = 1024
NB = 98
NBT = 99
NPAD = NB * BINSZ
NW = 32
EPW = 102400
WIN = 2048
NWIN = EPW // WIN
EPAD = NW * EPW
REG = 103888
CH = 128
HP = 128
BINNED_LEN = NW * REG + CH

_mesh = plsc.VectorSubcoreMesh(core_axis_name="c", subcore_axis_name="s")
_sc_params = pltpu.CompilerParams(needs_layout_passes=False)


@functools.partial(
    pl.kernel,
    out_type=(
        jax.ShapeDtypeStruct((BINNED_LEN,), jnp.int32),
        jax.ShapeDtypeStruct((NW * 128,), jnp.int32),
        jax.ShapeDtypeStruct((NW * 128,), jnp.int32),
    ),
    mesh=_mesh,
    scratch_types=[
        pltpu.VMEM((WIN,), jnp.int32),
        pltpu.VMEM((WIN,), jnp.int32),
        pltpu.VMEM((16 * 128,), jnp.int32),
        pltpu.VMEM((128,), jnp.int32),
        pltpu.VMEM((128,), jnp.int32),
        pltpu.VMEM((16 * 128,), jnp.int32),
        pltpu.VMEM((16,), jnp.int32),
        pltpu.VMEM((REG,), jnp.int32),
    ],
    compiler_params=_sc_params,
)
def _bin_edges(dst_hbm, src_hbm, binned_hbm, cntm_hbm, basem_hbm,
               dstwin, srcwin, cnt2, cnt1, bases, ptr2, tmpv, sorted_buf):
    w = lax.axis_index("s") * 2 + lax.axis_index("c")
    ebase = w * EPW
    iota = lax.iota(jnp.int32, 16)
    lane_row = iota * 128
    zeros16 = jnp.zeros((16,), jnp.int32)
    ones16 = jnp.ones((16,), jnp.int32)
    lane0 = iota == 0

    def zero_body(j, _):
        cnt2[pl.ds(j * 16, 16)] = zeros16
        return 0
    lax.fori_loop(0, 128, zero_body, 0)

    def hist_win(win, _):
        hstart = pl.multiple_of(ebase + win * WIN, 8)
        pltpu.sync_copy(dst_hbm.at[pl.ds(hstart, WIN)], dstwin)
        for k in range(WIN // 16):
            d = dstwin[pl.ds(k * 16, 16)]
            bins = lax.shift_right_logical(d, 10)
            plsc.addupdate_scatter(cnt2, [lane_row + bins], ones16)
        return 0
    lax.fori_loop(0, NWIN, hist_win, 0)

    def pfx(bi, p):
        fb = jnp.full((16,), bi, jnp.int32)
        cb = plsc.load_gather(cnt2, [lane_row + fb])
        tmpv[...] = cb
        ex = zeros16
        for lp in range(15):
            t = plsc.load_gather(tmpv, [jnp.full((16,), lp, jnp.int32)])
            ex = ex + jnp.where(iota > lp, t, zeros16)
        tot = (ex + cb)[15]
        plsc.store_scatter(bases, [fb], jnp.full((16,), p, jnp.int32),
                           mask=lane0)
        plsc.store_scatter(cnt1, [fb], jnp.full((16,), tot, jnp.int32),
                           mask=lane0)
        plsc.store_scatter(ptr2, [lane_row + fb],
                           jnp.full((16,), p, jnp.int32) + ex)
        return jnp.bitwise_and(p + tot + 15, -16)
    lax.fori_loop(0, NBT, pfx, 0)

    def scat_win(win, _):
        wstart = pl.multiple_of(ebase + win * WIN, 8)
        pltpu.sync_copy(dst_hbm.at[pl.ds(wstart, WIN)], dstwin)
        pltpu.sync_copy(src_hbm.at[pl.ds(wstart, WIN)], srcwin)

        for k in range(WIN // 16):
            d = dstwin[pl.ds(k * 16, 16)]
            sv = srcwin[pl.ds(k * 16, 16)]
            bins = lax.shift_right_logical(d, 10)
            dloc = jnp.bitwise_and(d, BINSZ - 1)
            val = jnp.bitwise_or(lax.shift_left(dloc, 17), sv)
            adr = lane_row + bins
            cur = plsc.load_gather(ptr2, [adr])
            plsc.store_scatter(ptr2, [adr], cur + 1)
            plsc.store_scatter(sorted_buf, [cur], val)
        return 0
    lax.fori_loop(0, NWIN, scat_win, 0)

    rstart = pl.multiple_of(w * REG, 8)
    pltpu.sync_copy(sorted_buf, binned_hbm.at[pl.ds(rstart, REG)])

    mstart = pl.multiple_of(w * 128, 8)
    pltpu.sync_copy(cnt1, cntm_hbm.at[pl.ds(mstart, 128)])
    pltpu.sync_copy(bases, basem_hbm.at[pl.ds(mstart, 128)])


@functools.partial(
    pl.kernel,
    out_type=(
        jax.ShapeDtypeStruct((NPAD * DP,), jnp.float32),
        jax.ShapeDtypeStruct((NPAD * DP,), jnp.float32),
        jax.ShapeDtypeStruct((NPAD * DP,), jnp.float32),
    ),
    mesh=_mesh,
    scratch_types=[
        pltpu.VMEM((NW * 128,), jnp.int32),
        pltpu.VMEM((NW * 128,), jnp.int32),
        pltpu.VMEM((CH,), jnp.int32),
        pltpu.VMEM((1, 128), jnp.int32),
        pltpu.VMEM((CH,), jnp.int32),
        pltpu.VMEM((CH, HP), jnp.float32),
        pltpu.VMEM((BINSZ * DP,), jnp.float32),
        pltpu.VMEM((BINSZ * DP,), jnp.float32),
        pltpu.VMEM((BINSZ * DP,), jnp.float32),
        pltpu.VMEM((BINSZ,), jnp.float32),
        pltpu.SemaphoreType.DMA,
    ],
    compiler_params=_sc_params,
)
def _aggregate(binned_hbm, hpad_hbm, cntm_hbm, basem_hbm,
               mean_hbm, max_hbm, min_hbm,
               cntv, basev, chunk, srcb, dstlb, rows,
               ssum, smax, smin, sdeg, sem):
    w = lax.axis_index("s") * 2 + lax.axis_index("c")
    iota = lax.iota(jnp.int32, 16)
    zf16 = jnp.zeros((16,), jnp.float32)
    neg = jnp.full((16,), -3e38, jnp.float32)
    posi = jnp.full((16,), 3e38, jnp.float32)
    ones = jnp.ones((16,), jnp.float32)
    lane0 = iota == 0

    pltpu.sync_copy(cntm_hbm, cntv)
    pltpu.sync_copy(basem_hbm, basev)

    def per_round(r, _):
        bi = w + r * NW

        @pl.when(bi < NB)
        def _():
            def zero_vec(j, _):
                ssum[pl.ds(j * 16, 16)] = zf16
                smax[pl.ds(j * 16, 16)] = neg
                smin[pl.ds(j * 16, 16)] = posi
                return 0
            lax.fori_loop(0, BINSZ * DP // 16, zero_vec, 0)

            def zero_deg(j, _):
                sdeg[pl.ds(j * 16, 16)] = zf16
                return 0
            lax.fori_loop(0, BINSZ // 16, zero_deg, 0)

            def per_src(sw, _):
                adr16 = jnp.full((16,), sw * 128 + bi, jnp.int32)
                cnt = plsc.load_gather(cntv, [adr16])[0]
                base = sw * REG + plsc.load_gather(basev, [adr16])[0]
                nch = (cnt + CH - 1) // CH

                def per_chunk(ci, _):
                    off = ci * CH
                    cstart = pl.multiple_of(base + off, 8)
                    pltpu.sync_copy(binned_hbm.at[pl.ds(cstart, CH)], chunk)

                    for k in range(CH // 16):
                        v = chunk[pl.ds(k * 16, 16)]
                        eid = off + k * 16 + iota
                        m = eid < cnt
                        sv = jnp.where(
                            m, jnp.bitwise_and(v, 0x1FFFF),
                            jnp.bitwise_and(eid, BINSZ - 1))
                        dv = jnp.bitwise_and(
                            lax.shift_right_logical(v, 17), BINSZ - 1)
                        srcb[0, pl.ds(k * 16, 16)] = sv
                        dstlb[pl.ds(k * 16, 16)] = dv

                    pltpu.async_copy(
                        hpad_hbm.at[srcb.at[0]], rows, sem).wait()

                    m_edges = jnp.minimum(CH, cnt - off)

                    def edge(i, _):
                        dlv = plsc.load_gather(dstlb,
                                               [jnp.full((16,), i, jnp.int32)])
                        ad = dlv[0] * DP
                        ml = rows[i, pl.ds(0, 16)]
                        mh = rows[i, pl.ds(16, 16)]
                        lo = pl.ds(ad, 16)
                        hi = pl.ds(ad + 16, 16)
                        ssum[lo] = ssum[lo] + ml
                        ssum[hi] = ssum[hi] + mh
                        smax[lo] = jnp.maximum(smax[lo], ml)
                        smax[hi] = jnp.maximum(smax[hi], mh)
                        smin[lo] = jnp.minimum(smin[lo], ml)
                        smin[hi] = jnp.minimum(smin[hi], mh)
                        plsc.addupdate_scatter(sdeg, [dlv], ones, mask=lane0)
                        return 0
                    lax.fori_loop(0, m_edges, edge, 0)
                    return 0
                lax.fori_loop(0, nch, per_chunk, 0)
                return 0
            lax.fori_loop(0, NW, per_src, 0)

            def fin_node(i, _):
                dgv = plsc.load_gather(sdeg, [jnp.full((16,), i, jnp.int32)])
                rcp = 1.0 / jnp.maximum(dgv, 1.0)
                has = dgv > 0.0
                ad = i * DP
                for half in (0, 16):
                    sl = pl.ds(ad + half, 16)
                    ssum[sl] = ssum[sl] * rcp
                    smax[sl] = jnp.where(has, smax[sl], zf16)
                    smin[sl] = jnp.where(has, smin[sl], zf16)
                return 0
            lax.fori_loop(0, BINSZ, fin_node, 0)

            obase = pl.multiple_of(bi * BINSZ * DP, 8)
            pltpu.sync_copy(ssum, mean_hbm.at[pl.ds(obase, BINSZ * DP)])
            pltpu.sync_copy(smax, max_hbm.at[pl.ds(obase, BINSZ * DP)])
            pltpu.sync_copy(smin, min_hbm.at[pl.ds(obase, BINSZ * DP)])
        return 0
    lax.fori_loop(0, (NB + NW - 1) // NW, per_round, 0)


def _tc1_body(mean_ref, max_ref, min_ref, wm_ref, wx_ref, wn_ref,
              bias_ref, snorm_ref, z_ref, sums_ref):
    i = pl.program_id(0)
    z = jnp.dot(mean_ref[...], wm_ref[...],
                preferred_element_type=jnp.float32)
    z += jnp.dot(max_ref[...], wx_ref[...],
                 preferred_element_type=jnp.float32)
    z += jnp.dot(min_ref[...], wn_ref[...],
                 preferred_element_type=jnp.float32)
    z = (z + bias_ref[...]) * snorm_ref[...]
    z_ref[...] = z
    rid = lax.broadcasted_iota(jnp.int32, z.shape, 0) + i * BINSZ
    zm = jnp.where(rid < N_NODES, z, 0.0)

    @pl.when(i == 0)
    def _():
        sums_ref[...] = jnp.zeros_like(sums_ref)

    sums_ref[0:1, 0:DP] += jnp.sum(zm, axis=0, keepdims=True)
    sums_ref[1:2, 0:DP] += jnp.sum(zm * zm, axis=0, keepdims=True)


def _tc2_body(z_ref, h_ref, sums_ref, g_ref, be_ref, o_ref):
    mu = sums_ref[0:1, 0:DP] / N_NODES
    ex2 = sums_ref[1:2, 0:DP] / N_NODES
    rstd = lax.rsqrt(ex2 - mu * mu + 1e-5)
    y = (z_ref[...] - mu) * rstd * g_ref[...] + be_ref[...]
    y = jnp.maximum(y, 0.0)
    o_ref[...] = h_ref[...] + y[:, 0:DIM]


def kernel(h, edge_index, e, snorm_n, W, b, gamma, beta):
    del e
    src = edge_index[0].astype(jnp.int32)
    dst = edge_index[1].astype(jnp.int32)
    npad_e = EPAD - N_EDGES
    dstp = jnp.concatenate([dst, jnp.full((npad_e,), NPAD, jnp.int32)])
    srcp = jnp.concatenate([src, jnp.zeros((npad_e,), jnp.int32)])
    hpad = jnp.pad(h, ((0, 0), (0, HP - DIM)))

    binned, cntm, basem = _bin_edges(dstp, srcp)
    mean_a, max_a, min_a = _aggregate(binned, hpad, cntm, basem)
    mean_a = mean_a.reshape(NPAD, DP)
    max_a = max_a.reshape(NPAD, DP)
    min_a = min_a.reshape(NPAD, DP)


    wm = jnp.pad(W[0:DIM], ((0, DP - DIM), (0, DP - DIM)))
    wx = jnp.pad(W[DIM:2 * DIM], ((0, DP - DIM), (0, DP - DIM)))
    wn = jnp.pad(W[2 * DIM:3 * DIM], ((0, DP - DIM), (0, DP - DIM)))
    bp = jnp.pad(b, (0, DP - DIM)).reshape(1, DP)
    gp = jnp.pad(gamma, (0, DP - DIM)).reshape(1, DP)
    bep = jnp.pad(beta, (0, DP - DIM)).reshape(1, DP)
    snp = jnp.pad(snorm_n, ((0, NPAD - N_NODES), (0, 0)))

    z, sums = pl.pallas_call(
        _tc1_body,
        grid=(NB,),
        in_specs=[
            pl.BlockSpec((BINSZ, DP), lambda i: (i, 0)),
            pl.BlockSpec((BINSZ, DP), lambda i: (i, 0)),
            pl.BlockSpec((BINSZ, DP), lambda i: (i, 0)),
            pl.BlockSpec((DP, DP), lambda i: (0, 0)),
            pl.BlockSpec((DP, DP), lambda i: (0, 0)),
            pl.BlockSpec((DP, DP), lambda i: (0, 0)),
            pl.BlockSpec((1, DP), lambda i: (0, 0)),
            pl.BlockSpec((BINSZ, 1), lambda i: (i, 0)),
        ],
        out_specs=[
            pl.BlockSpec((BINSZ, DP), lambda i: (i, 0)),
            pl.BlockSpec((8, 128), lambda i: (0, 0)),
        ],
        out_shape=[
            jax.ShapeDtypeStruct((NPAD, DP), jnp.float32),
            jax.ShapeDtypeStruct((8, 128), jnp.float32),
        ],
    )(mean_a, max_a, min_a, wm, wx, wn, bp, snp)

    nb2 = 1000
    out = pl.pallas_call(
        _tc2_body,
        grid=(N_NODES // nb2,),
        in_specs=[
            pl.BlockSpec((nb2, DP), lambda i: (i, 0)),
            pl.BlockSpec((nb2, DIM), lambda i: (i, 0)),
            pl.BlockSpec((8, 128), lambda i: (0, 0)),
            pl.BlockSpec((1, DP), lambda i: (0, 0)),
            pl.BlockSpec((1, DP), lambda i: (0, 0)),
        ],
        out_specs=pl.BlockSpec((nb2, DIM), lambda i: (i, 0)),
        out_shape=jax.ShapeDtypeStruct((N_NODES, DIM), jnp.float32),
    )(z, h, sums, gp, bep)
    return out
